# baseline (speedup 1.0000x reference)
import functools

import jax
import jax.numpy as jnp
from jax import lax
from jax.experimental import pallas as pl
from jax.experimental.pallas import tpu as pltpu
from jax.experimental.pallas import tpu_sc as plsc

_N = 10000
_E = 320000
_DIN = 128
_DH = 64
_NC = 2
_NS = 16
_NW = _NC * _NS
_EP = _E // _NW
_CH = 128
_NCH = 80
_EPAD = _NCH * _CH
_NB = 4
_NPAD = 10240
_SEG = _NPAD // _NS
_DW = 16

_mesh = plsc.VectorSubcoreMesh(core_axis_name="c", subcore_axis_name="s")


@functools.partial(
    pl.kernel,
    out_type=jax.ShapeDtypeStruct((_NC, _NPAD, _DW), jnp.float32),
    mesh=_mesh,
    scratch_types=[
        pltpu.VMEM((_NCH, _CH), jnp.int32),
        pltpu.VMEM((_CH, _DW), jnp.float32),
        pltpu.VMEM_SHARED((_NPAD, _DW), jnp.float32),
        pltpu.SemaphoreType.DMA,
    ],
    compiler_params=pltpu.CompilerParams(use_tc_tiling_on_sc=False),
)
def _degree(colp_hbm, ones_hbm, zeros_hbm, degp_hbm, coli_v, ones_v, acc_sh,
            ssem):
    cid = lax.axis_index("c")
    sid = lax.axis_index("s")
    gid = cid * _NS + sid
    pltpu.sync_copy(zeros_hbm, acc_sh.at[pl.ds(sid * _SEG, _SEG)])
    pltpu.sync_copy(colp_hbm.at[gid], coli_v)
    pltpu.sync_copy(ones_hbm, ones_v)
    plsc.subcore_barrier()

    @pl.loop(0, _NCH)
    def _fire(j):
        pltpu.async_copy(ones_v, acc_sh.at[coli_v.at[j]], ssem, add=True)

    @pl.loop(0, _NCH)
    def _drain(j):
        pltpu.make_async_copy(ones_v, acc_sh.at[coli_v.at[j]], ssem).wait()

    plsc.subcore_barrier()
    pltpu.sync_copy(acc_sh.at[pl.ds(sid * _SEG, _SEG)],
                    degp_hbm.at[cid, pl.ds(sid * _SEG, _SEG)])


@functools.partial(
    pl.kernel,
    out_type=jax.ShapeDtypeStruct((_NC, _NPAD, _DH), jnp.float32),
    mesh=_mesh,
    scratch_types=[
        pltpu.VMEM((_EPAD,), jnp.int32),
        pltpu.VMEM((_NCH, _CH), jnp.int32),
        pltpu.VMEM((_NB, _CH, _DH), jnp.float32),
        pltpu.VMEM_SHARED((_NPAD, _DH), jnp.float32),
        [pltpu.SemaphoreType.DMA] * _NB,
        [pltpu.SemaphoreType.DMA] * _NB,
    ],
    compiler_params=pltpu.CompilerParams(use_tc_tiling_on_sc=False),
)
def _propagate(g_hbm, rowp_hbm, colp_hbm, zeros_hbm, out_hbm,
               rowi_v, coli_v, buf_v, acc_sh, gsems, ssems):
    cid = lax.axis_index("c")
    sid = lax.axis_index("s")
    gid = cid * _NS + sid
    pltpu.sync_copy(zeros_hbm, acc_sh.at[pl.ds(sid * _SEG, _SEG)])
    pltpu.sync_copy(rowp_hbm.at[gid], rowi_v)
    pltpu.sync_copy(colp_hbm.at[gid], coli_v)
    plsc.subcore_barrier()

    def _gather(chunk, b):
        return pltpu.make_async_copy(
            g_hbm.at[rowi_v.at[pl.ds(chunk * _CH, _CH)]],
            buf_v.at[b], gsems[b])

    def _scatter(chunk, b):
        return pltpu.make_async_copy(
            buf_v.at[b], acc_sh.at[coli_v.at[chunk]], ssems[b])

    _gather(0, 0).start()
    _gather(1, 1).start()

    @pl.loop(0, _NCH, step=_NB)
    def _steady(j):
        for b in range(_NB):
            t = j + b
            _gather(t, b).wait()
            pltpu.async_copy(buf_v.at[b],
                             acc_sh.at[coli_v.at[t]], ssems[b], add=True)
            b2 = (b + 2) % _NB

            @pl.when(t >= 2)
            def _wait_prev():
                _scatter(t - 2, b2).wait()

            @pl.when(t + 2 < _NCH)
            def _next_gather():
                _gather(t + 2, b2).start()

    _scatter(_NCH - 2, (_NCH - 2) % _NB).wait()
    _scatter(_NCH - 1, (_NCH - 1) % _NB).wait()
    plsc.subcore_barrier()
    pltpu.sync_copy(acc_sh.at[pl.ds(sid * _SEG, _SEG)],
                    out_hbm.at[cid, pl.ds(sid * _SEG, _SEG)])


def _dis_from_degp(degp_ref):
    deg = degp_ref[0, : _N, 0:1] + degp_ref[1, : _N, 0:1] + 1.0
    return lax.rsqrt(deg)


def _tc_in_body(degp_ref, x_ref, w1_ref, g1_ref):
    dis = _dis_from_degp(degp_ref)
    h = jnp.dot(x_ref[:, :], w1_ref[:, :], preferred_element_type=jnp.float32)
    g1_ref[:, :] = h * dis


def _tc_mid_body(degp_ref, sp_ref, g1_ref, b1_ref, w2_ref, g2_ref):
    dis = _dis_from_degp(degp_ref)
    s = sp_ref[0, : _N, :] + sp_ref[1, : _N, :] + g1_ref[:, :]
    h1 = jnp.maximum(dis * s + b1_ref[:], 0.0)
    g2_ref[:, :] = jnp.dot(h1, w2_ref[:, :],
                           preferred_element_type=jnp.float32) * dis


def _tc_head_body(degp_ref, sp_ref, g2_ref, b2_ref, wc1_ref, bc1_ref,
                  wc2_ref, bc2_ref, out_ref):
    dis = _dis_from_degp(degp_ref)
    h2 = dis * (sp_ref[0, : _N, :] + sp_ref[1, : _N, :] + g2_ref[:, :]) + b2_ref[:]
    gm = jnp.mean(h2, axis=0, keepdims=True)
    gx = jnp.max(h2, axis=0, keepdims=True)
    rep = jnp.concatenate([gm, gx], axis=1)
    z = jnp.maximum(
        jnp.dot(rep, wc1_ref[:, :], preferred_element_type=jnp.float32)
        + bc1_ref[:], 0.0)
    o = (jnp.dot(z, wc2_ref[:, :], preferred_element_type=jnp.float32)
         + bc2_ref[:])
    out_ref[:, :] = jax.nn.sigmoid(o)


_tc_in = pl.pallas_call(
    _tc_in_body, out_shape=jax.ShapeDtypeStruct((_N, _DH), jnp.float32))
_tc_mid = pl.pallas_call(
    _tc_mid_body, out_shape=jax.ShapeDtypeStruct((_N, _DH), jnp.float32))
_tc_head = pl.pallas_call(
    _tc_head_body, out_shape=jax.ShapeDtypeStruct((1, 1), jnp.float32))


def kernel(x, edge_index, W1, b1, W2, b2, Wc1, bc1, Wc2, bc2):
    row = edge_index[0].reshape(_NW, _EP)
    col = edge_index[1].reshape(_NW, _EP)
    npadc = _EPAD - _EP
    padrows = _N + (jnp.arange(npadc, dtype=jnp.int32) % (_NPAD - _N))
    rowp = jnp.concatenate(
        [row, jnp.zeros((_NW, npadc), jnp.int32)], axis=1)
    colp = jnp.concatenate(
        [col, jnp.broadcast_to(padrows, (_NW, npadc))], axis=1)
    colp3 = colp.reshape(_NW, _NCH, _CH)
    ones_dw = jnp.ones((_CH, _DW), jnp.float32)
    zeros_dw = jnp.zeros((_SEG, _DW), jnp.float32)
    zeros_dh = jnp.zeros((_SEG, _DH), jnp.float32)

    degp = _degree(colp3, ones_dw, zeros_dw)
    g1 = _tc_in(degp, x, W1)
    s1p = _propagate(g1, rowp, colp3, zeros_dh)
    g2 = _tc_mid(degp, s1p, g1, b1, W2)
    s2p = _propagate(g2, rowp, colp3, zeros_dh)
    return _tc_head(degp, s2p, g2, b2, Wc1, bc1, Wc2, bc2)


# speedup vs baseline: 1.1200x; 1.1200x over previous
import functools

import jax
import jax.numpy as jnp
from jax import lax
from jax.experimental import pallas as pl
from jax.experimental.pallas import tpu as pltpu
from jax.experimental.pallas import tpu_sc as plsc

_N = 10000
_E = 320000
_DIN = 128
_DH = 64
_NC = 2
_NS = 16
_NW = _NC * _NS
_EP = _E // _NW
_CH = 128
_NCH = -(-_EP // _CH)
_EPAD = _NCH * _CH
_NPAD = 10240
_SEG = _NPAD // _NS
_DW = 16

_mesh = plsc.VectorSubcoreMesh(core_axis_name="c", subcore_axis_name="s")


@functools.partial(
    pl.kernel,
    out_type=jax.ShapeDtypeStruct((_NC, _NPAD, _DW), jnp.float32),
    mesh=_mesh,
    scratch_types=[
        pltpu.VMEM((_NCH, _CH), jnp.int32),
        pltpu.VMEM((_CH, _DW), jnp.float32),
        pltpu.VMEM_SHARED((_NPAD, _DW), jnp.float32),
        pltpu.SemaphoreType.DMA,
    ],
    compiler_params=pltpu.CompilerParams(use_tc_tiling_on_sc=False),
)
def _degree(colp_hbm, ones_hbm, zeros_hbm, degp_hbm, coli_v, ones_v, acc_sh,
            ssem):
    cid = lax.axis_index("c")
    sid = lax.axis_index("s")
    gid = cid * _NS + sid
    pltpu.sync_copy(zeros_hbm, acc_sh.at[pl.ds(sid * _SEG, _SEG)])
    pltpu.sync_copy(colp_hbm.at[gid], coli_v)
    pltpu.sync_copy(ones_hbm, ones_v)
    plsc.subcore_barrier()

    @pl.loop(0, _NCH)
    def _fire(j):
        pltpu.async_copy(ones_v, acc_sh.at[coli_v.at[j]], ssem, add=True)

    @pl.loop(0, _NCH)
    def _drain(j):
        pltpu.make_async_copy(ones_v, acc_sh.at[coli_v.at[j]], ssem).wait()

    plsc.subcore_barrier()
    pltpu.sync_copy(acc_sh.at[pl.ds(sid * _SEG, _SEG)],
                    degp_hbm.at[cid, pl.ds(sid * _SEG, _SEG)])


@functools.partial(
    pl.kernel,
    out_type=jax.ShapeDtypeStruct((_NC, _NPAD, _DH), jnp.float32),
    mesh=_mesh,
    scratch_types=[
        pltpu.VMEM((_EPAD,), jnp.int32),
        pltpu.VMEM((_NCH, _CH), jnp.int32),
        pltpu.VMEM((_CH, _DH), jnp.float32),
        pltpu.VMEM_SHARED((_NPAD, _DH), jnp.float32),
        pltpu.SemaphoreType.DMA,
    ],
    compiler_params=pltpu.CompilerParams(use_tc_tiling_on_sc=False),
)
def _propagate(g_hbm, rowp_hbm, colp_hbm, zeros_hbm, out_hbm,
               rowi_v, coli_v, buf_v, acc_sh, gsem):
    cid = lax.axis_index("c")
    sid = lax.axis_index("s")
    gid = cid * _NS + sid
    pltpu.sync_copy(zeros_hbm, acc_sh.at[pl.ds(sid * _SEG, _SEG)])
    pltpu.sync_copy(rowp_hbm.at[gid], rowi_v)
    pltpu.sync_copy(colp_hbm.at[gid], coli_v)
    plsc.subcore_barrier()

    @pl.loop(0, _NCH)
    def _step(i):
        pltpu.async_copy(g_hbm.at[rowi_v.at[pl.ds(i * _CH, _CH)]],
                         buf_v, gsem).wait()
        pltpu.sync_copy(buf_v, acc_sh.at[coli_v.at[i]], add=True)

    plsc.subcore_barrier()
    pltpu.sync_copy(acc_sh.at[pl.ds(sid * _SEG, _SEG)],
                    out_hbm.at[cid, pl.ds(sid * _SEG, _SEG)])


def _dis_from_degp(degp_ref):
    deg = degp_ref[0, : _N, 0:1] + degp_ref[1, : _N, 0:1] + 1.0
    return lax.rsqrt(deg)


def _tc_mm_body(x_ref, w1_ref, h_ref):
    h_ref[:, :] = jnp.dot(x_ref[:, :], w1_ref[:, :],
                          preferred_element_type=jnp.float32)


def _tc_scale_body(degp_ref, h_ref, g1_ref):
    g1_ref[:, :] = h_ref[:, :] * _dis_from_degp(degp_ref)


def _tc_mid_body(degp_ref, sp_ref, g1_ref, b1_ref, w2_ref, g2_ref):
    dis = _dis_from_degp(degp_ref)
    s = sp_ref[0, : _N, :] + sp_ref[1, : _N, :] + g1_ref[:, :]
    h1 = jnp.maximum(dis * s + b1_ref[:], 0.0)
    g2_ref[:, :] = jnp.dot(h1, w2_ref[:, :],
                           preferred_element_type=jnp.float32) * dis


def _tc_head_body(degp_ref, sp_ref, g2_ref, b2_ref, wc1_ref, bc1_ref,
                  wc2_ref, bc2_ref, out_ref):
    dis = _dis_from_degp(degp_ref)
    h2 = dis * (sp_ref[0, : _N, :] + sp_ref[1, : _N, :] + g2_ref[:, :]) + b2_ref[:]
    gm = jnp.mean(h2, axis=0, keepdims=True)
    gx = jnp.max(h2, axis=0, keepdims=True)
    rep = jnp.concatenate([gm, gx], axis=1)
    z = jnp.maximum(
        jnp.dot(rep, wc1_ref[:, :], preferred_element_type=jnp.float32)
        + bc1_ref[:], 0.0)
    o = (jnp.dot(z, wc2_ref[:, :], preferred_element_type=jnp.float32)
         + bc2_ref[:])
    out_ref[:, :] = jax.nn.sigmoid(o)


_tc_mm = pl.pallas_call(
    _tc_mm_body, out_shape=jax.ShapeDtypeStruct((_N, _DH), jnp.float32))
_tc_scale = pl.pallas_call(
    _tc_scale_body, out_shape=jax.ShapeDtypeStruct((_N, _DH), jnp.float32))
_tc_mid = pl.pallas_call(
    _tc_mid_body, out_shape=jax.ShapeDtypeStruct((_N, _DH), jnp.float32))
_tc_head = pl.pallas_call(
    _tc_head_body, out_shape=jax.ShapeDtypeStruct((1, 1), jnp.float32))


def kernel(x, edge_index, W1, b1, W2, b2, Wc1, bc1, Wc2, bc2):
    row = edge_index[0].reshape(_NW, _EP)
    col = edge_index[1].reshape(_NW, _EP)
    npadc = _EPAD - _EP
    padrows = _N + (jnp.arange(npadc, dtype=jnp.int32) % (_NPAD - _N))
    rowp = jnp.concatenate(
        [row, jnp.zeros((_NW, npadc), jnp.int32)], axis=1)
    colp = jnp.concatenate(
        [col, jnp.broadcast_to(padrows, (_NW, npadc))], axis=1)
    colp3 = colp.reshape(_NW, _NCH, _CH)
    ones_dw = jnp.ones((_CH, _DW), jnp.float32)
    zeros_dw = jnp.zeros((_SEG, _DW), jnp.float32)
    zeros_dh = jnp.zeros((_SEG, _DH), jnp.float32)

    h1raw = _tc_mm(x, W1)
    degp = _degree(colp3, ones_dw, zeros_dw)
    g1 = _tc_scale(degp, h1raw)
    s1p = _propagate(g1, rowp, colp3, zeros_dh)
    g2 = _tc_mid(degp, s1p, g1, b1, W2)
    s2p = _propagate(g2, rowp, colp3, zeros_dh)
    return _tc_head(degp, s2p, g2, b2, Wc1, bc1, Wc2, bc2)


# speedup vs baseline: 1.2606x; 1.1255x over previous
import functools

import jax
import jax.numpy as jnp
from jax import lax
from jax.experimental import pallas as pl
from jax.experimental.pallas import tpu as pltpu
from jax.experimental.pallas import tpu_sc as plsc

_N = 10000
_E = 320000
_DIN = 128
_DH = 64
_NC = 2
_NS = 16
_NW = _NC * _NS
_EP = _E // _NW
_CH = 128
_NCH = -(-_EP // _CH)
_EPAD = _NCH * _CH
_NPAD = 10240
_SEG = _NPAD // _NS
_DW = 16

_mesh = plsc.VectorSubcoreMesh(core_axis_name="c", subcore_axis_name="s")


@functools.partial(
    pl.kernel,
    out_type=jax.ShapeDtypeStruct((_NC, _NPAD, _DW), jnp.float32),
    mesh=_mesh,
    scratch_types=[
        pltpu.VMEM((_NCH, _CH), jnp.int32),
        pltpu.VMEM((_CH, _DW), jnp.float32),
        pltpu.VMEM_SHARED((_NPAD, _DW), jnp.float32),
        pltpu.SemaphoreType.DMA,
    ],
    compiler_params=pltpu.CompilerParams(use_tc_tiling_on_sc=False),
)
def _degree(colp_hbm, ones_hbm, zeros_hbm, degp_hbm, coli_v, ones_v, acc_sh,
            ssem):
    cid = lax.axis_index("c")
    sid = lax.axis_index("s")
    gid = cid * _NS + sid
    pltpu.sync_copy(zeros_hbm, acc_sh.at[pl.ds(sid * _SEG, _SEG)])
    pltpu.sync_copy(colp_hbm.at[gid], coli_v)
    pltpu.sync_copy(ones_hbm, ones_v)
    plsc.subcore_barrier()

    @pl.loop(0, _NCH)
    def _fire(j):
        pltpu.async_copy(ones_v, acc_sh.at[coli_v.at[j]], ssem, add=True)

    @pl.loop(0, _NCH)
    def _drain(j):
        pltpu.make_async_copy(ones_v, acc_sh.at[coli_v.at[j]], ssem).wait()

    plsc.subcore_barrier()
    pltpu.sync_copy(acc_sh.at[pl.ds(sid * _SEG, _SEG)],
                    degp_hbm.at[cid, pl.ds(sid * _SEG, _SEG)])


@functools.partial(
    pl.kernel,
    out_type=jax.ShapeDtypeStruct((_NC, _NPAD, _DH), jnp.float32),
    mesh=_mesh,
    scratch_types=[
        pltpu.VMEM((_EPAD,), jnp.int32),
        pltpu.VMEM((_NCH, _CH), jnp.int32),
        pltpu.VMEM((2, _CH, _DH), jnp.float32),
        pltpu.VMEM_SHARED((_NPAD, _DH), jnp.float32),
        pltpu.SemaphoreType.DMA,
        [pltpu.SemaphoreType.DMA] * 2,
    ],
    compiler_params=pltpu.CompilerParams(use_tc_tiling_on_sc=False),
)
def _propagate(g_hbm, rowp_hbm, colp_hbm, zeros_hbm, out_hbm,
               rowi_v, coli_v, buf_v, acc_sh, gsem, ssems):
    cid = lax.axis_index("c")
    sid = lax.axis_index("s")
    gid = cid * _NS + sid
    pltpu.sync_copy(zeros_hbm, acc_sh.at[pl.ds(sid * _SEG, _SEG)])
    pltpu.sync_copy(rowp_hbm.at[gid], rowi_v)
    pltpu.sync_copy(colp_hbm.at[gid], coli_v)
    plsc.subcore_barrier()

    def _scatter(chunk, b):
        return pltpu.make_async_copy(
            buf_v.at[b], acc_sh.at[coli_v.at[chunk]], ssems[b])

    @pl.loop(0, _NCH - 1, step=2)
    def _step(j):
        for b in range(2):
            t = j + b
            pltpu.async_copy(g_hbm.at[rowi_v.at[pl.ds(t * _CH, _CH)]],
                             buf_v.at[b], gsem).wait()

            @pl.when(t >= 1)
            def _wait_prev():
                _scatter(t - 1, 1 - b).wait()

            _scatter(t, b).start(add=True)

    pltpu.async_copy(g_hbm.at[rowi_v.at[pl.ds((_NCH - 1) * _CH, _CH)]],
                     buf_v.at[0], gsem).wait()
    _scatter(_NCH - 2, 1).wait()
    _scatter(_NCH - 1, 0).start(add=True)
    _scatter(_NCH - 1, 0).wait()
    plsc.subcore_barrier()
    pltpu.sync_copy(acc_sh.at[pl.ds(sid * _SEG, _SEG)],
                    out_hbm.at[cid, pl.ds(sid * _SEG, _SEG)])


def _dis_from_degp(degp_ref):
    deg = degp_ref[0, : _N, 0:1] + degp_ref[1, : _N, 0:1] + 1.0
    return lax.rsqrt(deg)


def _tc_in_body(degp_ref, x_ref, w1_ref, g1_ref):
    dis = _dis_from_degp(degp_ref)
    h = jnp.dot(x_ref[:, :], w1_ref[:, :], preferred_element_type=jnp.float32)
    g1_ref[:, :] = h * dis


def _tc_mid_body(degp_ref, sp_ref, g1_ref, b1_ref, w2_ref, g2_ref):
    dis = _dis_from_degp(degp_ref)
    s = sp_ref[0, : _N, :] + sp_ref[1, : _N, :] + g1_ref[:, :]
    h1 = jnp.maximum(dis * s + b1_ref[:], 0.0)
    g2_ref[:, :] = jnp.dot(h1, w2_ref[:, :],
                           preferred_element_type=jnp.float32) * dis


def _tc_head_body(degp_ref, sp_ref, g2_ref, b2_ref, wc1_ref, bc1_ref,
                  wc2_ref, bc2_ref, out_ref):
    dis = _dis_from_degp(degp_ref)
    h2 = dis * (sp_ref[0, : _N, :] + sp_ref[1, : _N, :] + g2_ref[:, :]) + b2_ref[:]
    gm = jnp.mean(h2, axis=0, keepdims=True)
    gx = jnp.max(h2, axis=0, keepdims=True)
    rep = jnp.concatenate([gm, gx], axis=1)
    z = jnp.maximum(
        jnp.dot(rep, wc1_ref[:, :], preferred_element_type=jnp.float32)
        + bc1_ref[:], 0.0)
    o = (jnp.dot(z, wc2_ref[:, :], preferred_element_type=jnp.float32)
         + bc2_ref[:])
    out_ref[:, :] = jax.nn.sigmoid(o)


_tc_in = pl.pallas_call(
    _tc_in_body, out_shape=jax.ShapeDtypeStruct((_N, _DH), jnp.float32))
_tc_mid = pl.pallas_call(
    _tc_mid_body, out_shape=jax.ShapeDtypeStruct((_N, _DH), jnp.float32))
_tc_head = pl.pallas_call(
    _tc_head_body, out_shape=jax.ShapeDtypeStruct((1, 1), jnp.float32))


def kernel(x, edge_index, W1, b1, W2, b2, Wc1, bc1, Wc2, bc2):
    row = edge_index[0].reshape(_NW, _EP)
    col = edge_index[1].reshape(_NW, _EP)
    npadc = _EPAD - _EP
    padrows = _N + (jnp.arange(npadc, dtype=jnp.int32) % (_NPAD - _N))
    rowp = jnp.concatenate(
        [row, jnp.zeros((_NW, npadc), jnp.int32)], axis=1)
    colp = jnp.concatenate(
        [col, jnp.broadcast_to(padrows, (_NW, npadc))], axis=1)
    colp3 = colp.reshape(_NW, _NCH, _CH)
    ones_dw = jnp.ones((_CH, _DW), jnp.float32)
    zeros_dw = jnp.zeros((_SEG, _DW), jnp.float32)
    zeros_dh = jnp.zeros((_SEG, _DH), jnp.float32)

    degp = _degree(colp3, ones_dw, zeros_dw)
    g1 = _tc_in(degp, x, W1)
    s1p = _propagate(g1, rowp, colp3, zeros_dh)
    g2 = _tc_mid(degp, s1p, g1, b1, W2)
    s2p = _propagate(g2, rowp, colp3, zeros_dh)
    return _tc_head(degp, s2p, g2, b2, Wc1, bc1, Wc2, bc2)
